# Initial kernel scaffold; baseline (speedup 1.0000x reference)
import jax
import jax.numpy as jnp
from jax.experimental import pallas as pl


def kernel(x, edge_index, edge_type, node_type, edge_filter, c1_Wr, c1_Wroot, c1_b, c2_Wr, c2_Wroot, c2_b, c3_Wr, c3_Wroot, c3_b, bil_W, bil_b, m1_W, m1_b, m2_W, m2_b, m3_W, m3_b):
    raise NotImplementedError("write your pallas kernel here")


# speedup vs baseline: 9.8350x; 9.8350x over previous
import functools

import jax
import jax.numpy as jnp
from jax import lax
from jax.experimental import pallas as pl
from jax.experimental.pallas import tpu as pltpu
from jax.experimental.pallas import tpu_sc as plsc

N = 10000
E = 320000
D = 128
R = 8
P = 4096

NC = 2
NS = 16
NW = NC * NS

EPW = E // NW
CH = 80
NCH = EPW // CH
NPAD = 10240
NPW = NPAD // NS
PPW = P // NW
SUP = 2000
NSUP = E // SUP

_mesh = plsc.VectorSubcoreMesh(core_axis_name="c", subcore_axis_name="s")
_sc_params = pltpu.CompilerParams(needs_layout_passes=False)


def _i32x16(v):
    return jnp.full((16,), v, jnp.int32)


@functools.partial(
    pl.kernel,
    out_type=(
        jax.ShapeDtypeStruct((E,), jnp.int32),
        jax.ShapeDtypeStruct((E,), jnp.float32),
        jax.ShapeDtypeStruct((P,), jnp.int32),
        jax.ShapeDtypeStruct((P,), jnp.int32),
    ),
    mesh=_mesh,
    scratch_types=[
        pltpu.VMEM((N * R,), jnp.int32),
        pltpu.VMEM((SUP,), jnp.int32),
        pltpu.VMEM((SUP,), jnp.int32),
        pltpu.VMEM((CH,), jnp.int32),
        pltpu.VMEM((CH,), jnp.int32),
        pltpu.VMEM((CH,), jnp.float32),
        pltpu.VMEM((16,), jnp.int32),
        pltpu.VMEM((PPW,), jnp.int32),
        pltpu.VMEM_SHARED((N * R,), jnp.int32),
        pltpu.VMEM_SHARED((16,), jnp.int32),
    ],
    compiler_params=_sc_params,
)
def _sc_prep(src_h, dst_h, et_h, nt_h, ef0_h, ef1_h,
             gidx_h, w_h, ia_h, ib_h,
             cnt_v, dsup, esup, schk, gchk, wchk, kv, pchk,
             cnt_s, k_s):
    cid = lax.axis_index("c")
    sid = lax.axis_index("s")
    wid = sid * NC + cid

    @pl.when(sid == 0)
    def _():
        def zbody(i, _):
            cnt_v[pl.ds(i * 16, 16)] = _i32x16(0)
            return 0
        lax.fori_loop(0, (N * R) // 16, zbody, 0)

        ones = _i32x16(1)

        def cbody(c, _):
            base = c * SUP
            pltpu.sync_copy(dst_h.at[pl.ds(base, SUP)], dsup)
            pltpu.sync_copy(et_h.at[pl.ds(base, SUP)], esup)

            def ibody(j, _):
                d = dsup[pl.ds(j * 16, 16)]
                e = esup[pl.ds(j * 16, 16)]
                plsc.addupdate_scatter(cnt_v, [d * R + e], ones)
                return 0
            lax.fori_loop(0, SUP // 16, ibody, 0)
            return 0
        lax.fori_loop(0, NSUP, cbody, 0)

        def kbody(c, acc):
            pltpu.sync_copy(nt_h.at[pl.ds(c * SUP, SUP)], dsup)

            def kinner(j, acc2):
                m = dsup[pl.ds(j * 16, 16)] == _i32x16(0)
                return acc2 + plsc.all_reduce_population_count(m)
            return lax.fori_loop(0, SUP // 16, kinner, acc)
        kacc = lax.fori_loop(0, N // SUP, kbody, _i32x16(0))
        km1 = kacc - 1
        km1 = jnp.where(km1 < 0, _i32x16(N - 1), km1)
        kv[...] = km1
        pltpu.sync_copy(kv, k_s)
        pltpu.sync_copy(cnt_v, cnt_s)

    plsc.subcore_barrier()
    pltpu.sync_copy(cnt_s, cnt_v)
    pltpu.sync_copy(k_s, kv)

    def wbody(c, _):
        base = wid * EPW + c * CH
        pltpu.sync_copy(src_h.at[pl.ds(base, CH)], schk)
        pltpu.sync_copy(dst_h.at[pl.ds(base, CH)], dsup.at[pl.ds(0, CH)])
        pltpu.sync_copy(et_h.at[pl.ds(base, CH)], esup.at[pl.ds(0, CH)])

        def inner(j, _):
            sl = pl.ds(j * 16, 16)
            s = schk[sl]
            d = dsup[sl]
            e = esup[sl]
            gchk[sl] = e * N + s
            cnt16 = plsc.load_gather(cnt_v, [d * R + e])
            cf = cnt16.astype(jnp.float32)
            wchk[sl] = 1.0 / jnp.maximum(cf, 1.0)
            return 0
        lax.fori_loop(0, CH // 16, inner, 0)
        pltpu.sync_copy(gchk, gidx_h.at[pl.ds(base, CH)])
        pltpu.sync_copy(wchk, w_h.at[pl.ds(base, CH)])
        return 0
    lax.fori_loop(0, NCH, wbody, 0)

    kvv = kv[...]
    for ef_h, o_h in ((ef0_h, ia_h), (ef1_h, ib_h)):
        pbase = wid * PPW
        pltpu.sync_copy(ef_h.at[pl.ds(pbase, PPW)], pchk)

        def pbody(j, _):
            sl = pl.ds(j * 16, 16)
            pchk[sl] = jnp.minimum(pchk[sl], kvv)
            return 0
        lax.fori_loop(0, PPW // 16, pbody, 0)
        pltpu.sync_copy(pchk, o_h.at[pl.ds(pbase, PPW)])


@functools.partial(
    pl.kernel,
    out_type=jax.ShapeDtypeStruct((NC, NPAD, D), jnp.float32),
    mesh=_mesh,
    scratch_types=[
        pltpu.VMEM((CH,), jnp.int32),
        pltpu.VMEM((CH,), jnp.int32),
        pltpu.VMEM((CH,), jnp.float32),
        pltpu.VMEM((CH, D), jnp.float32),
        pltpu.VMEM_SHARED((NPAD, D), jnp.float32),
        pltpu.SemaphoreType.DMA,
    ],
    compiler_params=_sc_params,
)
def _sc_scatter(y_h, gidx_h, dst_h, w_h, z_h,
                out_h,
                gbuf, dbuf, wbuf, rows, acc, sem):
    cid = lax.axis_index("c")
    sid = lax.axis_index("s")
    wid = sid * NC + cid

    pltpu.sync_copy(z_h, acc.at[pl.ds(sid * NPW, NPW)])
    plsc.subcore_barrier()

    def cbody(c, _):
        base = wid * EPW + c * CH
        pltpu.sync_copy(gidx_h.at[pl.ds(base, CH)], gbuf)
        pltpu.sync_copy(dst_h.at[pl.ds(base, CH)], dbuf)
        pltpu.sync_copy(w_h.at[pl.ds(base, CH)], wbuf)
        pltpu.async_copy(y_h.at[gbuf], rows, sem).wait()

        def ebody(e, _):
            wv = plsc.load_gather(wbuf, [_i32x16(0) + e])
            rv = rows.at[e]
            for j in range(D // 16):
                sl = pl.ds(j * 16, 16)
                rv[sl] = rv[sl] * wv
            return 0
        lax.fori_loop(0, CH, ebody, 0)
        pltpu.sync_copy(rows, acc.at[dbuf], add=True)
        return 0
    lax.fori_loop(0, NCH, cbody, 0)

    plsc.subcore_barrier()
    sl = pl.ds(sid * NPW, NPW)
    pltpu.sync_copy(acc.at[sl], out_h.at[cid].at[sl])


@functools.partial(
    pl.kernel,
    out_type=jax.ShapeDtypeStruct((2, P, D), jnp.float32),
    mesh=_mesh,
    scratch_types=[
        pltpu.VMEM((PPW,), jnp.int32),
        pltpu.VMEM((PPW, D), jnp.float32),
        pltpu.SemaphoreType.DMA,
    ],
    compiler_params=_sc_params,
)
def _sc_pairs(h_h, ia_h, ib_h, out_h, ibuf, rows, sem):
    cid = lax.axis_index("c")
    sid = lax.axis_index("s")
    wid = sid * NC + cid
    base = wid * PPW
    for side, idx_h in ((0, ia_h), (1, ib_h)):
        pltpu.sync_copy(idx_h.at[pl.ds(base, PPW)], ibuf)
        pltpu.async_copy(h_h.at[ibuf], rows, sem).wait()
        pltpu.sync_copy(rows, out_h.at[side].at[pl.ds(base, PPW)])


BN = 1000


def _mm9_first_body(x_ref, w_ref, root_ref, y_ref):
    r = pl.program_id(1)
    h = x_ref[...]
    out = jnp.dot(h, w_ref[0], preferred_element_type=jnp.float32)

    @pl.when(r == 0)
    def _():
        root_ref[...] = out

    @pl.when(r > 0)
    def _():
        y_ref[0] = out


def _mm9_next_body(root_ref, b_ref, msg_ref, w_ref, nroot_ref, y_ref):
    r = pl.program_id(1)
    h = jax.nn.relu(root_ref[...] + b_ref[...] + msg_ref[0] + msg_ref[1])
    out = jnp.dot(h, w_ref[0], preferred_element_type=jnp.float32)

    @pl.when(r == 0)
    def _():
        nroot_ref[...] = out

    @pl.when(r > 0)
    def _():
        y_ref[0] = out


def _mm9_out_specs():
    return [
        pl.BlockSpec((BN, D), lambda i, r: (i, 0)),
        pl.BlockSpec((1, BN, D), lambda i, r: (jnp.maximum(r - 1, 0), i, 0)),
    ]


def _mm9_out_shapes():
    return [
        jax.ShapeDtypeStruct((N, D), jnp.float32),
        jax.ShapeDtypeStruct((R, N, D), jnp.float32),
    ]


_mm9_first = pl.pallas_call(
    _mm9_first_body,
    grid=(N // BN, R + 1),
    in_specs=[
        pl.BlockSpec((BN, D), lambda i, r: (i, 0)),
        pl.BlockSpec((1, D, D), lambda i, r: (r, 0, 0)),
    ],
    out_specs=_mm9_out_specs(),
    out_shape=_mm9_out_shapes(),
)

_mm9_next = pl.pallas_call(
    _mm9_next_body,
    grid=(N // BN, R + 1),
    in_specs=[
        pl.BlockSpec((BN, D), lambda i, r: (i, 0)),
        pl.BlockSpec((1, D), lambda i, r: (0, 0)),
        pl.BlockSpec((2, BN, D), lambda i, r: (0, i, 0)),
        pl.BlockSpec((1, D, D), lambda i, r: (r, 0, 0)),
    ],
    out_specs=_mm9_out_specs(),
    out_shape=_mm9_out_shapes(),
)


def _combine_body(root_ref, b_ref, msg_ref, h_ref):
    h_ref[...] = jax.nn.relu(root_ref[...] + b_ref[...] + msg_ref[0] + msg_ref[1])


_combine = pl.pallas_call(
    _combine_body,
    grid=(N // BN,),
    in_specs=[
        pl.BlockSpec((BN, D), lambda i: (i, 0)),
        pl.BlockSpec((1, D), lambda i: (0, 0)),
        pl.BlockSpec((2, BN, D), lambda i: (0, i, 0)),
    ],
    out_specs=pl.BlockSpec((BN, D), lambda i: (i, 0)),
    out_shape=jax.ShapeDtypeStruct((N, D), jnp.float32),
)


BP = 64


def _bil_body(fa_ref, fb_ref, w2_ref, bb_ref,
              m1w_ref, m1b_ref, m2w_ref, m2b_ref, m3w_ref, m3b_ref,
              out_ref):
    fa = fa_ref[...]
    fb = fb_ref[...]
    t = jnp.dot(fa.astype(jnp.bfloat16), w2_ref[...],
                preferred_element_type=jnp.float32)
    t3 = t.reshape(BP, D, D)
    s = jnp.sum(t3 * fb[:, None, :], axis=2)
    x1 = jax.nn.relu(s + bb_ref[...])
    x2 = jax.nn.relu(jnp.dot(x1, m1w_ref[...],
                             preferred_element_type=jnp.float32) + m1b_ref[...])
    x3 = jax.nn.relu(jnp.dot(x2, m2w_ref[...],
                             preferred_element_type=jnp.float32) + m2b_ref[...])
    out_ref[...] = jnp.dot(x3, m3w_ref[...],
                           preferred_element_type=jnp.float32) + m3b_ref[...]


_bilmlp = pl.pallas_call(
    _bil_body,
    grid=(P // BP,),
    in_specs=[
        pl.BlockSpec((BP, D), lambda i: (i, 0)),
        pl.BlockSpec((BP, D), lambda i: (i, 0)),
        pl.BlockSpec((D, D * D), lambda i: (0, 0)),
        pl.BlockSpec((1, D), lambda i: (0, 0)),
        pl.BlockSpec((D, D), lambda i: (0, 0)),
        pl.BlockSpec((1, D), lambda i: (0, 0)),
        pl.BlockSpec((D, D), lambda i: (0, 0)),
        pl.BlockSpec((1, D), lambda i: (0, 0)),
        pl.BlockSpec((D, D), lambda i: (0, 0)),
        pl.BlockSpec((1, D), lambda i: (0, 0)),
    ],
    out_specs=pl.BlockSpec((BP, D), lambda i: (i, 0)),
    out_shape=jax.ShapeDtypeStruct((P, D), jnp.float32),
)


def kernel(x, edge_index, edge_type, node_type, edge_filter,
           c1_Wr, c1_Wroot, c1_b, c2_Wr, c2_Wroot, c2_b,
           c3_Wr, c3_Wroot, c3_b,
           bil_W, bil_b, m1_W, m1_b, m2_W, m2_b, m3_W, m3_b):
    src = edge_index[0]
    dst = edge_index[1]

    gidx, w, ia, ib = _sc_prep(src, dst, edge_type, node_type,
                               edge_filter[0], edge_filter[1])

    zrows = jnp.zeros((NPW, D), jnp.float32)

    w1 = jnp.concatenate([c1_Wroot[None], c1_Wr], axis=0)
    w2 = jnp.concatenate([c2_Wroot[None], c2_Wr], axis=0)
    w3 = jnp.concatenate([c3_Wroot[None], c3_Wr], axis=0)

    root1, y1 = _mm9_first(x, w1)
    m1 = _sc_scatter(y1.reshape(R * N, D), gidx, dst, w, zrows)
    root2, y2 = _mm9_next(root1, c1_b.reshape(1, D), m1, w2)
    m2 = _sc_scatter(y2.reshape(R * N, D), gidx, dst, w, zrows)
    root3, y3 = _mm9_next(root2, c2_b.reshape(1, D), m2, w3)
    m3 = _sc_scatter(y3.reshape(R * N, D), gidx, dst, w, zrows)
    h3 = _combine(root3, c3_b.reshape(1, D), m3)

    pairs = _sc_pairs(h3, ia, ib)

    w2t = jnp.transpose(bil_W, (1, 0, 2)).reshape(D, D * D).astype(jnp.bfloat16)
    return _bilmlp(pairs[0], pairs[1], w2t,
                   bil_b.reshape(1, D),
                   m1_W, m1_b.reshape(1, D),
                   m2_W, m2_b.reshape(1, D),
                   m3_W, m3_b.reshape(1, D))


# speedup vs baseline: 22.9693x; 2.3355x over previous
import functools

import jax
import jax.numpy as jnp
from jax import lax
from jax.experimental import pallas as pl
from jax.experimental.pallas import tpu as pltpu
from jax.experimental.pallas import tpu_sc as plsc

N = 10000
E = 320000
D = 128
R = 8
P = 4096

NC = 2
NS = 16
NW = NC * NS

EPW = E // NW
CH = 80
NCH = EPW // CH
NPAD = 10240
NPW = NPAD // NS
PPW = P // NW
SUP = 2000
NSUP = E // SUP

_mesh = plsc.VectorSubcoreMesh(core_axis_name="c", subcore_axis_name="s")
_sc_params = pltpu.CompilerParams(needs_layout_passes=False)


def _i32x16(v):
    return jnp.full((16,), v, jnp.int32)


NRR = 640
EPS = E // NS
NSUPC = EPS // SUP


@functools.partial(
    pl.kernel,
    out_type=(
        jax.ShapeDtypeStruct((E,), jnp.int32),
        jax.ShapeDtypeStruct((E,), jnp.float32),
        jax.ShapeDtypeStruct((P,), jnp.int32),
        jax.ShapeDtypeStruct((P,), jnp.int32),
    ),
    mesh=_mesh,
    scratch_types=[
        pltpu.VMEM((NRR, 128), jnp.int32),
        pltpu.VMEM((SUP,), jnp.int32),
        pltpu.VMEM((SUP,), jnp.int32),
        pltpu.VMEM((SUP,), jnp.int32),
        pltpu.VMEM((SUP,), jnp.int32),
        pltpu.VMEM((SUP,), jnp.float32),
        pltpu.VMEM((128,), jnp.int32),
        pltpu.VMEM((16,), jnp.int32),
        pltpu.VMEM((PPW,), jnp.int32),
        pltpu.VMEM_SHARED((NRR, 128), jnp.int32),
        pltpu.VMEM_SHARED((16,), jnp.int32),
    ],
    compiler_params=_sc_params,
)
def _sc_prep(src_h, dst_h, et_h, nt_h, ef0_h, ef1_h, zc_h,
             gidx_h, w_h, ia_h, ib_h,
             cnt_v, ssup, dsup, esup, gbuf, wbuf, rix, kv, pchk,
             cnt_s, k_s):
    cid = lax.axis_index("c")
    sid = lax.axis_index("s")
    wid = sid * NC + cid

    pltpu.sync_copy(zc_h, cnt_v)
    nrw = NRR // NS
    zsl = pl.ds(sid * nrw, nrw)
    pltpu.sync_copy(zc_h.at[zsl], cnt_s.at[zsl])

    ones = _i32x16(1)

    def cbody(c, _):
        base = sid * EPS + c * SUP
        pltpu.sync_copy(dst_h.at[pl.ds(base, SUP)], dsup)
        pltpu.sync_copy(et_h.at[pl.ds(base, SUP)], esup)

        def ibody(j, _):
            d = dsup[pl.ds(j * 16, 16)]
            e = esup[pl.ds(j * 16, 16)]
            b2 = d * R + e
            plsc.addupdate_scatter(cnt_v, [b2 >> 7, b2 & 127], ones)
            return 0
        lax.fori_loop(0, SUP // 16, ibody, 0, unroll=4)
        return 0
    lax.fori_loop(0, NSUPC, cbody, 0)

    @pl.when(sid == 0)
    def _():
        def kbody(c, acc):
            pltpu.sync_copy(nt_h.at[pl.ds(c * SUP, SUP)], dsup)

            def kinner(j, acc2):
                m = dsup[pl.ds(j * 16, 16)] == _i32x16(0)
                return acc2 + plsc.all_reduce_population_count(m)
            return lax.fori_loop(0, SUP // 16, kinner, acc)
        kacc = lax.fori_loop(0, N // SUP, kbody, _i32x16(0))
        km1 = kacc - 1
        km1 = jnp.where(km1 < 0, _i32x16(N - 1), km1)
        kv[...] = km1
        pltpu.sync_copy(kv, k_s)

    plsc.subcore_barrier()

    iota16 = lax.iota(jnp.int32, 16)

    def mbody(c, _):
        for j in range(8):
            rix[pl.ds(j * 16, 16)] = iota16 + (c * 128 + j * 16)
        pltpu.sync_copy(cnt_v.at[pl.ds(c * 128, 128)], cnt_s.at[rix], add=True)
        return 0
    lax.fori_loop(0, NRR // 128, mbody, 0)

    plsc.subcore_barrier()
    pltpu.sync_copy(cnt_s, cnt_v)
    pltpu.sync_copy(k_s, kv)

    def wbody(c, _):
        base = wid * EPW + c * SUP
        pltpu.sync_copy(src_h.at[pl.ds(base, SUP)], ssup)
        pltpu.sync_copy(dst_h.at[pl.ds(base, SUP)], dsup)
        pltpu.sync_copy(et_h.at[pl.ds(base, SUP)], esup)

        def inner(j, _):
            sl = pl.ds(j * 16, 16)
            s = ssup[sl]
            d = dsup[sl]
            e = esup[sl]
            gbuf[sl] = e * N + s
            b2 = d * R + e
            cnt16 = plsc.load_gather(cnt_v, [b2 >> 7, b2 & 127])
            cf = cnt16.astype(jnp.float32)
            wbuf[sl] = 1.0 / jnp.maximum(cf, 1.0)
            return 0
        lax.fori_loop(0, SUP // 16, inner, 0, unroll=4)
        pltpu.sync_copy(gbuf, gidx_h.at[pl.ds(base, SUP)])
        pltpu.sync_copy(wbuf, w_h.at[pl.ds(base, SUP)])
        return 0
    lax.fori_loop(0, EPW // SUP, wbody, 0)

    kvv = kv[...]
    for ef_h, o_h in ((ef0_h, ia_h), (ef1_h, ib_h)):
        pbase = wid * PPW
        pltpu.sync_copy(ef_h.at[pl.ds(pbase, PPW)], pchk)

        def pbody(j, _):
            sl = pl.ds(j * 16, 16)
            pchk[sl] = jnp.minimum(pchk[sl], kvv)
            return 0
        lax.fori_loop(0, PPW // 16, pbody, 0)
        pltpu.sync_copy(pchk, o_h.at[pl.ds(pbase, PPW)])


@functools.partial(
    pl.kernel,
    out_type=jax.ShapeDtypeStruct((NC, NPAD, D), jnp.float32),
    mesh=_mesh,
    scratch_types=[
        pltpu.VMEM((SUP,), jnp.int32),
        pltpu.VMEM((SUP,), jnp.int32),
        pltpu.VMEM((SUP,), jnp.float32),
        pltpu.VMEM((CH,), jnp.int32),
        pltpu.VMEM((CH,), jnp.int32),
        pltpu.VMEM((CH, D), jnp.float32),
        pltpu.VMEM((CH, D), jnp.float32),
        pltpu.SemaphoreType.DMA,
        pltpu.SemaphoreType.DMA,
        pltpu.SemaphoreType.DMA,
        pltpu.SemaphoreType.DMA,
        pltpu.VMEM_SHARED((NPAD, D), jnp.float32),
    ],
    compiler_params=_sc_params,
)
def _sc_scatter(y_h, gidx_h, dst_h, w_h, z_h,
                out_h,
                gw, dw, ww, dbuf0, dbuf1, rows0, rows1,
                gsem0, gsem1, ssem0, ssem1, acc):
    cid = lax.axis_index("c")
    sid = lax.axis_index("s")
    wid = sid * NC + cid
    ebase = wid * EPW
    WCH = SUP // CH

    pltpu.sync_copy(z_h, acc.at[pl.ds(sid * NPW, NPW)])
    plsc.subcore_barrier()

    def start_gather(c, rows, gsem):
        pltpu.async_copy(y_h.at[gw.at[pl.ds(c * CH, CH)]], rows, gsem)

    def wait_gather(rows, gsem):
        pltpu.make_async_copy(y_h.at[gw.at[pl.ds(0, CH)]], rows, gsem).wait()

    def wait_scatter(rows, dbuf, ssem):
        pltpu.make_async_copy(rows, acc.at[dbuf], ssem).wait()

    def process(c, rows, dbuf, ssem):
        for j in range(CH // 16):
            dbuf[pl.ds(j * 16, 16)] = dw[pl.ds(c * CH + j * 16, 16)]

        def ebody(e, _):
            wv = plsc.load_gather(ww, [_i32x16(0) + (c * CH + e)])
            rv = rows.at[e]
            for j in range(D // 16):
                sl = pl.ds(j * 16, 16)
                rv[sl] = rv[sl] * wv
            return 0
        lax.fori_loop(0, CH, ebody, 0, unroll=4)
        pltpu.async_copy(rows, acc.at[dbuf], ssem, add=True)

    def wloop(s, _):
        wbase = ebase + s * SUP
        pltpu.sync_copy(gidx_h.at[pl.ds(wbase, SUP)], gw)
        pltpu.sync_copy(dst_h.at[pl.ds(wbase, SUP)], dw)
        pltpu.sync_copy(w_h.at[pl.ds(wbase, SUP)], ww)
        start_gather(0, rows0, gsem0)

        def pbody(k, _):
            c0 = 2 * k
            wait_gather(rows0, gsem0)

            @pl.when(k > 0)
            def _():
                wait_scatter(rows1, dbuf1, ssem1)
            start_gather(c0 + 1, rows1, gsem1)
            process(c0, rows0, dbuf0, ssem0)

            wait_gather(rows1, gsem1)
            wait_scatter(rows0, dbuf0, ssem0)
            start_gather(c0 + 2, rows0, gsem0)
            process(c0 + 1, rows1, dbuf1, ssem1)
            return 0
        lax.fori_loop(0, WCH // 2, pbody, 0)

        wait_gather(rows0, gsem0)
        wait_scatter(rows1, dbuf1, ssem1)
        process(WCH - 1, rows0, dbuf0, ssem0)
        wait_scatter(rows0, dbuf0, ssem0)
        return 0
    lax.fori_loop(0, EPW // SUP, wloop, 0)

    plsc.subcore_barrier()
    sl = pl.ds(sid * NPW, NPW)
    pltpu.sync_copy(acc.at[sl], out_h.at[cid].at[sl])


@functools.partial(
    pl.kernel,
    out_type=jax.ShapeDtypeStruct((2, P, D), jnp.float32),
    mesh=_mesh,
    scratch_types=[
        pltpu.VMEM((PPW,), jnp.int32),
        pltpu.VMEM((PPW, D), jnp.float32),
        pltpu.SemaphoreType.DMA,
    ],
    compiler_params=_sc_params,
)
def _sc_pairs(h_h, ia_h, ib_h, out_h, ibuf, rows, sem):
    cid = lax.axis_index("c")
    sid = lax.axis_index("s")
    wid = sid * NC + cid
    base = wid * PPW
    for side, idx_h in ((0, ia_h), (1, ib_h)):
        pltpu.sync_copy(idx_h.at[pl.ds(base, PPW)], ibuf)
        pltpu.async_copy(h_h.at[ibuf], rows, sem).wait()
        pltpu.sync_copy(rows, out_h.at[side].at[pl.ds(base, PPW)])


BN = 1000


def _mm9_first_body(x_ref, w_ref, root_ref, y_ref):
    r = pl.program_id(1)
    h = x_ref[...]
    out = jnp.dot(h, w_ref[0], preferred_element_type=jnp.float32)

    @pl.when(r == 0)
    def _():
        root_ref[...] = out

    @pl.when(r > 0)
    def _():
        y_ref[0] = out


def _mm9_next_body(root_ref, b_ref, msg_ref, w_ref, nroot_ref, y_ref):
    r = pl.program_id(1)
    h = jax.nn.relu(root_ref[...] + b_ref[...] + msg_ref[0] + msg_ref[1])
    out = jnp.dot(h, w_ref[0], preferred_element_type=jnp.float32)

    @pl.when(r == 0)
    def _():
        nroot_ref[...] = out

    @pl.when(r > 0)
    def _():
        y_ref[0] = out


def _mm9_out_specs():
    return [
        pl.BlockSpec((BN, D), lambda i, r: (i, 0)),
        pl.BlockSpec((1, BN, D), lambda i, r: (jnp.maximum(r - 1, 0), i, 0)),
    ]


def _mm9_out_shapes():
    return [
        jax.ShapeDtypeStruct((N, D), jnp.float32),
        jax.ShapeDtypeStruct((R, N, D), jnp.float32),
    ]


_mm9_first = pl.pallas_call(
    _mm9_first_body,
    grid=(N // BN, R + 1),
    in_specs=[
        pl.BlockSpec((BN, D), lambda i, r: (i, 0)),
        pl.BlockSpec((1, D, D), lambda i, r: (r, 0, 0)),
    ],
    out_specs=_mm9_out_specs(),
    out_shape=_mm9_out_shapes(),
)

_mm9_next = pl.pallas_call(
    _mm9_next_body,
    grid=(N // BN, R + 1),
    in_specs=[
        pl.BlockSpec((BN, D), lambda i, r: (i, 0)),
        pl.BlockSpec((1, D), lambda i, r: (0, 0)),
        pl.BlockSpec((2, BN, D), lambda i, r: (0, i, 0)),
        pl.BlockSpec((1, D, D), lambda i, r: (r, 0, 0)),
    ],
    out_specs=_mm9_out_specs(),
    out_shape=_mm9_out_shapes(),
)


def _combine_body(root_ref, b_ref, msg_ref, h_ref):
    h_ref[...] = jax.nn.relu(root_ref[...] + b_ref[...] + msg_ref[0] + msg_ref[1])


_combine = pl.pallas_call(
    _combine_body,
    grid=(N // BN,),
    in_specs=[
        pl.BlockSpec((BN, D), lambda i: (i, 0)),
        pl.BlockSpec((1, D), lambda i: (0, 0)),
        pl.BlockSpec((2, BN, D), lambda i: (0, i, 0)),
    ],
    out_specs=pl.BlockSpec((BN, D), lambda i: (i, 0)),
    out_shape=jax.ShapeDtypeStruct((N, D), jnp.float32),
)


BP = 64


def _bil_body(fa_ref, fb_ref, w2_ref, bb_ref,
              m1w_ref, m1b_ref, m2w_ref, m2b_ref, m3w_ref, m3b_ref,
              out_ref):
    fa = fa_ref[...]
    fb = fb_ref[...]
    t = jnp.dot(fa.astype(jnp.bfloat16), w2_ref[...],
                preferred_element_type=jnp.float32)
    t3 = t.reshape(BP, D, D)
    s = jnp.sum(t3 * fb[:, None, :], axis=2)
    x1 = jax.nn.relu(s + bb_ref[...])
    x2 = jax.nn.relu(jnp.dot(x1, m1w_ref[...],
                             preferred_element_type=jnp.float32) + m1b_ref[...])
    x3 = jax.nn.relu(jnp.dot(x2, m2w_ref[...],
                             preferred_element_type=jnp.float32) + m2b_ref[...])
    out_ref[...] = jnp.dot(x3, m3w_ref[...],
                           preferred_element_type=jnp.float32) + m3b_ref[...]


_bilmlp = pl.pallas_call(
    _bil_body,
    grid=(P // BP,),
    in_specs=[
        pl.BlockSpec((BP, D), lambda i: (i, 0)),
        pl.BlockSpec((BP, D), lambda i: (i, 0)),
        pl.BlockSpec((D, D * D), lambda i: (0, 0)),
        pl.BlockSpec((1, D), lambda i: (0, 0)),
        pl.BlockSpec((D, D), lambda i: (0, 0)),
        pl.BlockSpec((1, D), lambda i: (0, 0)),
        pl.BlockSpec((D, D), lambda i: (0, 0)),
        pl.BlockSpec((1, D), lambda i: (0, 0)),
        pl.BlockSpec((D, D), lambda i: (0, 0)),
        pl.BlockSpec((1, D), lambda i: (0, 0)),
    ],
    out_specs=pl.BlockSpec((BP, D), lambda i: (i, 0)),
    out_shape=jax.ShapeDtypeStruct((P, D), jnp.float32),
)


def kernel(x, edge_index, edge_type, node_type, edge_filter,
           c1_Wr, c1_Wroot, c1_b, c2_Wr, c2_Wroot, c2_b,
           c3_Wr, c3_Wroot, c3_b,
           bil_W, bil_b, m1_W, m1_b, m2_W, m2_b, m3_W, m3_b):
    src = edge_index[0]
    dst = edge_index[1]

    zcnt = jnp.zeros((NRR, 128), jnp.int32)
    gidx, w, ia, ib = _sc_prep(src, dst, edge_type, node_type,
                               edge_filter[0], edge_filter[1], zcnt)

    zrows = jnp.zeros((NPW, D), jnp.float32)

    w1 = jnp.concatenate([c1_Wroot[None], c1_Wr], axis=0)
    w2 = jnp.concatenate([c2_Wroot[None], c2_Wr], axis=0)
    w3 = jnp.concatenate([c3_Wroot[None], c3_Wr], axis=0)

    root1, y1 = _mm9_first(x, w1)
    m1 = _sc_scatter(y1.reshape(R * N, D), gidx, dst, w, zrows)
    root2, y2 = _mm9_next(root1, c1_b.reshape(1, D), m1, w2)
    m2 = _sc_scatter(y2.reshape(R * N, D), gidx, dst, w, zrows)
    root3, y3 = _mm9_next(root2, c2_b.reshape(1, D), m2, w3)
    m3 = _sc_scatter(y3.reshape(R * N, D), gidx, dst, w, zrows)
    h3 = _combine(root3, c3_b.reshape(1, D), m3)

    pairs = _sc_pairs(h3, ia, ib)

    w2t = jnp.transpose(bil_W, (1, 0, 2)).reshape(D, D * D).astype(jnp.bfloat16)
    return _bilmlp(pairs[0], pairs[1], w2t,
                   bil_b.reshape(1, D),
                   m1_W, m1_b.reshape(1, D),
                   m2_W, m2_b.reshape(1, D),
                   m3_W, m3_b.reshape(1, D))


# speedup vs baseline: 24.8538x; 1.0820x over previous
import functools

import jax
import jax.numpy as jnp
from jax import lax
from jax.experimental import pallas as pl
from jax.experimental.pallas import tpu as pltpu
from jax.experimental.pallas import tpu_sc as plsc

N = 10000
E = 320000
D = 128
R = 8
P = 4096

NC = 2
NS = 16
NW = NC * NS

EPW = E // NW
CH = 80
NCH = EPW // CH
NPAD = 10240
NPW = NPAD // NS
PPW = P // NW
SUP = 2000
NSUP = E // SUP

_mesh = plsc.VectorSubcoreMesh(core_axis_name="c", subcore_axis_name="s")
_sc_params = pltpu.CompilerParams(needs_layout_passes=False)


def _i32x16(v):
    return jnp.full((16,), v, jnp.int32)


NRR = 640
EPS = E // NS
NSUPC = EPS // SUP


@functools.partial(
    pl.kernel,
    out_type=(
        jax.ShapeDtypeStruct((E,), jnp.int32),
        jax.ShapeDtypeStruct((E,), jnp.float32),
        jax.ShapeDtypeStruct((P,), jnp.int32),
        jax.ShapeDtypeStruct((P,), jnp.int32),
    ),
    mesh=_mesh,
    scratch_types=[
        pltpu.VMEM((NRR, 128), jnp.int32),
        pltpu.VMEM((SUP,), jnp.int32),
        pltpu.VMEM((SUP,), jnp.int32),
        pltpu.VMEM((SUP,), jnp.int32),
        pltpu.VMEM((SUP,), jnp.int32),
        pltpu.VMEM((SUP,), jnp.float32),
        pltpu.VMEM((128,), jnp.int32),
        pltpu.VMEM((16,), jnp.int32),
        pltpu.VMEM((PPW,), jnp.int32),
        pltpu.VMEM_SHARED((NRR, 128), jnp.int32),
        pltpu.VMEM_SHARED((16,), jnp.int32),
    ],
    compiler_params=_sc_params,
)
def _sc_prep(src_h, dst_h, et_h, nt_h, ef0_h, ef1_h, zc_h,
             gidx_h, w_h, ia_h, ib_h,
             cnt_v, ssup, dsup, esup, gbuf, wbuf, rix, kv, pchk,
             cnt_s, k_s):
    cid = lax.axis_index("c")
    sid = lax.axis_index("s")
    wid = sid * NC + cid

    pltpu.sync_copy(zc_h, cnt_v)
    nrw = NRR // NS
    zsl = pl.ds(sid * nrw, nrw)
    pltpu.sync_copy(zc_h.at[zsl], cnt_s.at[zsl])

    ones = _i32x16(1)

    def cbody(c, _):
        base = sid * EPS + c * SUP
        pltpu.sync_copy(dst_h.at[pl.ds(base, SUP)], dsup)
        pltpu.sync_copy(et_h.at[pl.ds(base, SUP)], esup)

        def ibody(j, _):
            d = dsup[pl.ds(j * 16, 16)]
            e = esup[pl.ds(j * 16, 16)]
            b2 = d * R + e
            plsc.addupdate_scatter(cnt_v, [b2 >> 7, b2 & 127], ones)
            return 0
        lax.fori_loop(0, SUP // 16, ibody, 0, unroll=4)
        return 0
    lax.fori_loop(0, NSUPC, cbody, 0)

    @pl.when(sid == 0)
    def _():
        def kbody(c, acc):
            pltpu.sync_copy(nt_h.at[pl.ds(c * SUP, SUP)], dsup)

            def kinner(j, acc2):
                m = dsup[pl.ds(j * 16, 16)] == _i32x16(0)
                return acc2 + plsc.all_reduce_population_count(m)
            return lax.fori_loop(0, SUP // 16, kinner, acc)
        kacc = lax.fori_loop(0, N // SUP, kbody, _i32x16(0))
        km1 = kacc - 1
        km1 = jnp.where(km1 < 0, _i32x16(N - 1), km1)
        kv[...] = km1
        pltpu.sync_copy(kv, k_s)

    plsc.subcore_barrier()

    iota16 = lax.iota(jnp.int32, 16)

    def mbody(c, _):
        for j in range(8):
            rix[pl.ds(j * 16, 16)] = iota16 + (c * 128 + j * 16)
        pltpu.sync_copy(cnt_v.at[pl.ds(c * 128, 128)], cnt_s.at[rix], add=True)
        return 0
    lax.fori_loop(0, NRR // 128, mbody, 0)

    plsc.subcore_barrier()
    pltpu.sync_copy(cnt_s, cnt_v)
    pltpu.sync_copy(k_s, kv)

    def wbody(c, _):
        base = wid * EPW + c * SUP
        pltpu.sync_copy(src_h.at[pl.ds(base, SUP)], ssup)
        pltpu.sync_copy(dst_h.at[pl.ds(base, SUP)], dsup)
        pltpu.sync_copy(et_h.at[pl.ds(base, SUP)], esup)

        def inner(j, _):
            sl = pl.ds(j * 16, 16)
            s = ssup[sl]
            d = dsup[sl]
            e = esup[sl]
            gbuf[sl] = e * N + s
            b2 = d * R + e
            cnt16 = plsc.load_gather(cnt_v, [b2 >> 7, b2 & 127])
            cf = cnt16.astype(jnp.float32)
            wbuf[sl] = 1.0 / jnp.maximum(cf, 1.0)
            return 0
        lax.fori_loop(0, SUP // 16, inner, 0, unroll=4)
        pltpu.sync_copy(gbuf, gidx_h.at[pl.ds(base, SUP)])
        pltpu.sync_copy(wbuf, w_h.at[pl.ds(base, SUP)])
        return 0
    lax.fori_loop(0, EPW // SUP, wbody, 0)

    kvv = kv[...]
    for ef_h, o_h in ((ef0_h, ia_h), (ef1_h, ib_h)):
        pbase = wid * PPW
        pltpu.sync_copy(ef_h.at[pl.ds(pbase, PPW)], pchk)

        def pbody(j, _):
            sl = pl.ds(j * 16, 16)
            pchk[sl] = jnp.minimum(pchk[sl], kvv)
            return 0
        lax.fori_loop(0, PPW // 16, pbody, 0)
        pltpu.sync_copy(pchk, o_h.at[pl.ds(pbase, PPW)])


@functools.partial(
    pl.kernel,
    out_type=jax.ShapeDtypeStruct((NC, NPAD, D), jnp.float32),
    mesh=_mesh,
    scratch_types=[
        pltpu.VMEM((SUP,), jnp.int32),
        pltpu.VMEM((SUP,), jnp.int32),
        pltpu.VMEM((SUP,), jnp.float32),
        pltpu.VMEM((CH,), jnp.int32),
        pltpu.VMEM((CH,), jnp.int32),
        pltpu.VMEM((CH, D), jnp.float32),
        pltpu.VMEM((CH, D), jnp.float32),
        pltpu.SemaphoreType.DMA,
        pltpu.SemaphoreType.DMA,
        pltpu.SemaphoreType.DMA,
        pltpu.SemaphoreType.DMA,
        pltpu.VMEM_SHARED((NPAD, D), jnp.float32),
    ],
    compiler_params=_sc_params,
)
def _sc_scatter(y_h, gidx_h, dst_h, w_h, z_h,
                out_h,
                gw, dw, ww, dbuf0, dbuf1, rows0, rows1,
                gsem0, gsem1, ssem0, ssem1, acc):
    cid = lax.axis_index("c")
    sid = lax.axis_index("s")
    wid = sid * NC + cid
    ebase = wid * EPW
    WCH = SUP // CH

    pltpu.sync_copy(z_h, acc.at[pl.ds(sid * NPW, NPW)])
    plsc.subcore_barrier()

    def start_gather(c, rows, gsem):
        pltpu.async_copy(y_h.at[gw.at[pl.ds(c * CH, CH)]], rows, gsem)

    def wait_gather(rows, gsem):
        pltpu.make_async_copy(y_h.at[gw.at[pl.ds(0, CH)]], rows, gsem).wait()

    def wait_scatter(rows, dbuf, ssem):
        pltpu.make_async_copy(rows, acc.at[dbuf], ssem).wait()

    def process(c, rows, dbuf, ssem):
        for j in range(CH // 16):
            dbuf[pl.ds(j * 16, 16)] = dw[pl.ds(c * CH + j * 16, 16)]

        @plsc.parallel_loop(0, CH, 1, unroll=4)
        def _(e):
            wv = plsc.load_gather(ww, [_i32x16(0) + (c * CH + e)])
            rv = rows.at[e]
            for j in range(D // 16):
                sl = pl.ds(j * 16, 16)
                rv[sl] = rv[sl] * wv
        pltpu.async_copy(rows, acc.at[dbuf], ssem, add=True)

    def wloop(s, _):
        wbase = ebase + s * SUP
        pltpu.sync_copy(gidx_h.at[pl.ds(wbase, SUP)], gw)
        pltpu.sync_copy(dst_h.at[pl.ds(wbase, SUP)], dw)
        pltpu.sync_copy(w_h.at[pl.ds(wbase, SUP)], ww)
        start_gather(0, rows0, gsem0)

        def pbody(k, _):
            c0 = 2 * k
            wait_gather(rows0, gsem0)

            @pl.when(k > 0)
            def _():
                wait_scatter(rows1, dbuf1, ssem1)
            start_gather(c0 + 1, rows1, gsem1)
            process(c0, rows0, dbuf0, ssem0)

            wait_gather(rows1, gsem1)
            wait_scatter(rows0, dbuf0, ssem0)
            start_gather(c0 + 2, rows0, gsem0)
            process(c0 + 1, rows1, dbuf1, ssem1)
            return 0
        lax.fori_loop(0, WCH // 2, pbody, 0)

        wait_gather(rows0, gsem0)
        wait_scatter(rows1, dbuf1, ssem1)
        process(WCH - 1, rows0, dbuf0, ssem0)
        wait_scatter(rows0, dbuf0, ssem0)
        return 0
    lax.fori_loop(0, EPW // SUP, wloop, 0)

    plsc.subcore_barrier()
    sl = pl.ds(sid * NPW, NPW)
    pltpu.sync_copy(acc.at[sl], out_h.at[cid].at[sl])


@functools.partial(
    pl.kernel,
    out_type=jax.ShapeDtypeStruct((2, P, D), jnp.float32),
    mesh=_mesh,
    scratch_types=[
        pltpu.VMEM((PPW,), jnp.int32),
        pltpu.VMEM((PPW, D), jnp.float32),
        pltpu.SemaphoreType.DMA,
    ],
    compiler_params=_sc_params,
)
def _sc_pairs(h_h, ia_h, ib_h, out_h, ibuf, rows, sem):
    cid = lax.axis_index("c")
    sid = lax.axis_index("s")
    wid = sid * NC + cid
    base = wid * PPW
    for side, idx_h in ((0, ia_h), (1, ib_h)):
        pltpu.sync_copy(idx_h.at[pl.ds(base, PPW)], ibuf)
        pltpu.async_copy(h_h.at[ibuf], rows, sem).wait()
        pltpu.sync_copy(rows, out_h.at[side].at[pl.ds(base, PPW)])


BN = 1000


def _mm9_first_body(x_ref, w_ref, root_ref, y_ref):
    r = pl.program_id(1)
    h = x_ref[...]
    out = jnp.dot(h, w_ref[0], preferred_element_type=jnp.float32)

    @pl.when(r == 0)
    def _():
        root_ref[...] = out

    @pl.when(r > 0)
    def _():
        y_ref[0] = out


def _mm9_next_body(root_ref, b_ref, msg_ref, w_ref, nroot_ref, y_ref):
    r = pl.program_id(1)
    h = jax.nn.relu(root_ref[...] + b_ref[...] + msg_ref[0] + msg_ref[1])
    out = jnp.dot(h, w_ref[0], preferred_element_type=jnp.float32)

    @pl.when(r == 0)
    def _():
        nroot_ref[...] = out

    @pl.when(r > 0)
    def _():
        y_ref[0] = out


def _mm9_out_specs():
    return [
        pl.BlockSpec((BN, D), lambda i, r: (i, 0)),
        pl.BlockSpec((1, BN, D), lambda i, r: (jnp.maximum(r - 1, 0), i, 0)),
    ]


def _mm9_out_shapes():
    return [
        jax.ShapeDtypeStruct((N, D), jnp.float32),
        jax.ShapeDtypeStruct((R, N, D), jnp.float32),
    ]


_mm9_first = pl.pallas_call(
    _mm9_first_body,
    grid=(N // BN, R + 1),
    in_specs=[
        pl.BlockSpec((BN, D), lambda i, r: (i, 0)),
        pl.BlockSpec((1, D, D), lambda i, r: (r, 0, 0)),
    ],
    out_specs=_mm9_out_specs(),
    out_shape=_mm9_out_shapes(),
)

_mm9_next = pl.pallas_call(
    _mm9_next_body,
    grid=(N // BN, R + 1),
    in_specs=[
        pl.BlockSpec((BN, D), lambda i, r: (i, 0)),
        pl.BlockSpec((1, D), lambda i, r: (0, 0)),
        pl.BlockSpec((2, BN, D), lambda i, r: (0, i, 0)),
        pl.BlockSpec((1, D, D), lambda i, r: (r, 0, 0)),
    ],
    out_specs=_mm9_out_specs(),
    out_shape=_mm9_out_shapes(),
)


def _combine_body(root_ref, b_ref, msg_ref, h_ref):
    h_ref[...] = jax.nn.relu(root_ref[...] + b_ref[...] + msg_ref[0] + msg_ref[1])


_combine = pl.pallas_call(
    _combine_body,
    grid=(N // BN,),
    in_specs=[
        pl.BlockSpec((BN, D), lambda i: (i, 0)),
        pl.BlockSpec((1, D), lambda i: (0, 0)),
        pl.BlockSpec((2, BN, D), lambda i: (0, i, 0)),
    ],
    out_specs=pl.BlockSpec((BN, D), lambda i: (i, 0)),
    out_shape=jax.ShapeDtypeStruct((N, D), jnp.float32),
)


BP = 256


def _bil_body(fa_ref, fb_ref, w2_ref, bb_ref,
              m1w_ref, m1b_ref, m2w_ref, m2b_ref, m3w_ref, m3b_ref,
              out_ref):
    fa = fa_ref[...]
    fb = fb_ref[...]
    t = jnp.dot(fa.astype(jnp.bfloat16), w2_ref[...],
                preferred_element_type=jnp.float32)
    t3 = t.reshape(BP, D, D)
    s = jnp.sum(t3 * fb[:, None, :], axis=2)
    x1 = jax.nn.relu(s + bb_ref[...])
    x2 = jax.nn.relu(jnp.dot(x1, m1w_ref[...],
                             preferred_element_type=jnp.float32) + m1b_ref[...])
    x3 = jax.nn.relu(jnp.dot(x2, m2w_ref[...],
                             preferred_element_type=jnp.float32) + m2b_ref[...])
    out_ref[...] = jnp.dot(x3, m3w_ref[...],
                           preferred_element_type=jnp.float32) + m3b_ref[...]


_bilmlp = pl.pallas_call(
    _bil_body,
    grid=(P // BP,),
    in_specs=[
        pl.BlockSpec((BP, D), lambda i: (i, 0)),
        pl.BlockSpec((BP, D), lambda i: (i, 0)),
        pl.BlockSpec((D, D * D), lambda i: (0, 0)),
        pl.BlockSpec((1, D), lambda i: (0, 0)),
        pl.BlockSpec((D, D), lambda i: (0, 0)),
        pl.BlockSpec((1, D), lambda i: (0, 0)),
        pl.BlockSpec((D, D), lambda i: (0, 0)),
        pl.BlockSpec((1, D), lambda i: (0, 0)),
        pl.BlockSpec((D, D), lambda i: (0, 0)),
        pl.BlockSpec((1, D), lambda i: (0, 0)),
    ],
    out_specs=pl.BlockSpec((BP, D), lambda i: (i, 0)),
    out_shape=jax.ShapeDtypeStruct((P, D), jnp.float32),
)


def kernel(x, edge_index, edge_type, node_type, edge_filter,
           c1_Wr, c1_Wroot, c1_b, c2_Wr, c2_Wroot, c2_b,
           c3_Wr, c3_Wroot, c3_b,
           bil_W, bil_b, m1_W, m1_b, m2_W, m2_b, m3_W, m3_b):
    src = edge_index[0]
    dst = edge_index[1]

    zcnt = jnp.zeros((NRR, 128), jnp.int32)
    gidx, w, ia, ib = _sc_prep(src, dst, edge_type, node_type,
                               edge_filter[0], edge_filter[1], zcnt)

    zrows = jnp.zeros((NPW, D), jnp.float32)

    w1 = jnp.concatenate([c1_Wroot[None], c1_Wr], axis=0)
    w2 = jnp.concatenate([c2_Wroot[None], c2_Wr], axis=0)
    w3 = jnp.concatenate([c3_Wroot[None], c3_Wr], axis=0)

    root1, y1 = _mm9_first(x, w1)
    m1 = _sc_scatter(y1.reshape(R * N, D), gidx, dst, w, zrows)
    root2, y2 = _mm9_next(root1, c1_b.reshape(1, D), m1, w2)
    m2 = _sc_scatter(y2.reshape(R * N, D), gidx, dst, w, zrows)
    root3, y3 = _mm9_next(root2, c2_b.reshape(1, D), m2, w3)
    m3 = _sc_scatter(y3.reshape(R * N, D), gidx, dst, w, zrows)
    h3 = _combine(root3, c3_b.reshape(1, D), m3)

    pairs = _sc_pairs(h3, ia, ib)

    w2t = jnp.transpose(bil_W, (1, 0, 2)).reshape(D, D * D).astype(jnp.bfloat16)
    return _bilmlp(pairs[0], pairs[1], w2t,
                   bil_b.reshape(1, D),
                   m1_W, m1_b.reshape(1, D),
                   m2_W, m2_b.reshape(1, D),
                   m3_W, m3_b.reshape(1, D))


# speedup vs baseline: 25.3665x; 1.0206x over previous
import functools

import jax
import jax.numpy as jnp
from jax import lax
from jax.experimental import pallas as pl
from jax.experimental.pallas import tpu as pltpu
from jax.experimental.pallas import tpu_sc as plsc

N = 10000
E = 320000
D = 128
R = 8
P = 4096

NC = 2
NS = 16
NW = NC * NS

EPW = E // NW
CH = 80
NCH = EPW // CH
NPAD = 10240
NPW = NPAD // NS
PPW = P // NW
SUP = 2000
NSUP = E // SUP

_mesh = plsc.VectorSubcoreMesh(core_axis_name="c", subcore_axis_name="s")
_sc_params = pltpu.CompilerParams(needs_layout_passes=False)


def _i32x16(v):
    return jnp.full((16,), v, jnp.int32)


NRR = 640
EPS = E // NS
NSUPC = EPS // SUP


@functools.partial(
    pl.kernel,
    out_type=(
        jax.ShapeDtypeStruct((E,), jnp.int32),
        jax.ShapeDtypeStruct((E,), jnp.float32),
        jax.ShapeDtypeStruct((P,), jnp.int32),
        jax.ShapeDtypeStruct((P,), jnp.int32),
    ),
    mesh=_mesh,
    scratch_types=[
        pltpu.VMEM((NRR, 128), jnp.int32),
        pltpu.VMEM((SUP,), jnp.int32),
        pltpu.VMEM((SUP,), jnp.int32),
        pltpu.VMEM((SUP,), jnp.int32),
        pltpu.VMEM((SUP,), jnp.int32),
        pltpu.VMEM((SUP,), jnp.float32),
        pltpu.VMEM((128,), jnp.int32),
        pltpu.VMEM((16,), jnp.int32),
        pltpu.VMEM((PPW,), jnp.int32),
        pltpu.VMEM_SHARED((NRR, 128), jnp.int32),
        pltpu.VMEM_SHARED((16,), jnp.int32),
    ],
    compiler_params=_sc_params,
)
def _sc_prep(src_h, dst_h, et_h, nt_h, ef0_h, ef1_h, zc_h,
             gidx_h, w_h, ia_h, ib_h,
             cnt_v, ssup, dsup, esup, gbuf, wbuf, rix, kv, pchk,
             cnt_s, k_s):
    cid = lax.axis_index("c")
    sid = lax.axis_index("s")
    wid = sid * NC + cid

    pltpu.sync_copy(zc_h, cnt_v)
    nrw = NRR // NS
    zsl = pl.ds(sid * nrw, nrw)
    pltpu.sync_copy(zc_h.at[zsl], cnt_s.at[zsl])

    ones = _i32x16(1)

    def cbody(c, _):
        base = sid * EPS + c * SUP
        pltpu.sync_copy(dst_h.at[pl.ds(base, SUP)], dsup)
        pltpu.sync_copy(et_h.at[pl.ds(base, SUP)], esup)

        def ibody(j, _):
            d = dsup[pl.ds(j * 16, 16)]
            e = esup[pl.ds(j * 16, 16)]
            b2 = d * R + e
            plsc.addupdate_scatter(cnt_v, [b2 >> 7, b2 & 127], ones)
            return 0
        lax.fori_loop(0, SUP // 16, ibody, 0, unroll=4)
        return 0
    lax.fori_loop(0, NSUPC, cbody, 0)

    @pl.when(sid == 0)
    def _():
        def kbody(c, acc):
            pltpu.sync_copy(nt_h.at[pl.ds(c * SUP, SUP)], dsup)

            def kinner(j, acc2):
                m = dsup[pl.ds(j * 16, 16)] == _i32x16(0)
                return acc2 + plsc.all_reduce_population_count(m)
            return lax.fori_loop(0, SUP // 16, kinner, acc)
        kacc = lax.fori_loop(0, N // SUP, kbody, _i32x16(0))
        km1 = kacc - 1
        km1 = jnp.where(km1 < 0, _i32x16(N - 1), km1)
        kv[...] = km1
        pltpu.sync_copy(kv, k_s)

    plsc.subcore_barrier()

    iota16 = lax.iota(jnp.int32, 16)

    def mbody(c, _):
        for j in range(8):
            rix[pl.ds(j * 16, 16)] = iota16 + (c * 128 + j * 16)
        pltpu.sync_copy(cnt_v.at[pl.ds(c * 128, 128)], cnt_s.at[rix], add=True)
        return 0
    lax.fori_loop(0, NRR // 128, mbody, 0)

    plsc.subcore_barrier()
    pltpu.sync_copy(cnt_s, cnt_v)
    pltpu.sync_copy(k_s, kv)

    def wbody(c, _):
        base = wid * EPW + c * SUP
        pltpu.sync_copy(src_h.at[pl.ds(base, SUP)], ssup)
        pltpu.sync_copy(dst_h.at[pl.ds(base, SUP)], dsup)
        pltpu.sync_copy(et_h.at[pl.ds(base, SUP)], esup)

        def inner(j, _):
            sl = pl.ds(j * 16, 16)
            s = ssup[sl]
            d = dsup[sl]
            e = esup[sl]
            gbuf[sl] = e * N + s
            b2 = d * R + e
            cnt16 = plsc.load_gather(cnt_v, [b2 >> 7, b2 & 127])
            cf = cnt16.astype(jnp.float32)
            wbuf[sl] = 1.0 / jnp.maximum(cf, 1.0)
            return 0
        lax.fori_loop(0, SUP // 16, inner, 0, unroll=4)
        pltpu.sync_copy(gbuf, gidx_h.at[pl.ds(base, SUP)])
        pltpu.sync_copy(wbuf, w_h.at[pl.ds(base, SUP)])
        return 0
    lax.fori_loop(0, EPW // SUP, wbody, 0)

    kvv = kv[...]
    for ef_h, o_h in ((ef0_h, ia_h), (ef1_h, ib_h)):
        pbase = wid * PPW
        pltpu.sync_copy(ef_h.at[pl.ds(pbase, PPW)], pchk)

        def pbody(j, _):
            sl = pl.ds(j * 16, 16)
            pchk[sl] = jnp.minimum(pchk[sl], kvv)
            return 0
        lax.fori_loop(0, PPW // 16, pbody, 0)
        pltpu.sync_copy(pchk, o_h.at[pl.ds(pbase, PPW)])


@functools.partial(
    pl.kernel,
    out_type=jax.ShapeDtypeStruct((NC, NPAD, D), jnp.float32),
    mesh=_mesh,
    scratch_types=[
        pltpu.VMEM((SUP,), jnp.int32),
        pltpu.VMEM((SUP,), jnp.int32),
        pltpu.VMEM((SUP,), jnp.int32),
        pltpu.VMEM((SUP,), jnp.int32),
        pltpu.VMEM((SUP,), jnp.float32),
        pltpu.VMEM((SUP,), jnp.float32),
        pltpu.VMEM((CH,), jnp.int32),
        pltpu.VMEM((CH,), jnp.int32),
        pltpu.VMEM((CH, D), jnp.float32),
        pltpu.VMEM((CH, D), jnp.float32),
        pltpu.SemaphoreType.DMA,
        pltpu.SemaphoreType.DMA,
        pltpu.SemaphoreType.DMA,
        pltpu.SemaphoreType.DMA,
        pltpu.SemaphoreType.DMA,
        pltpu.VMEM_SHARED((NPAD, D), jnp.float32),
    ],
    compiler_params=_sc_params,
)
def _sc_scatter(y_h, gidx_h, dst_h, w_h, z_h,
                out_h,
                gwA, gwB, dwA, dwB, wwA, wwB, dbuf0, dbuf1, rows0, rows1,
                gsem0, gsem1, ssem0, ssem1, wsem, acc):
    gw2 = (gwA, gwB)
    dw2 = (dwA, dwB)
    ww2 = (wwA, wwB)
    cid = lax.axis_index("c")
    sid = lax.axis_index("s")
    wid = sid * NC + cid
    ebase = wid * EPW
    WCH = SUP // CH
    NWIN = EPW // SUP

    pltpu.sync_copy(z_h, acc.at[pl.ds(sid * NPW, NPW)])

    def load_window(s, b):
        wbase = ebase + s * SUP
        pltpu.async_copy(gidx_h.at[pl.ds(wbase, SUP)], gw2[b], wsem)
        pltpu.async_copy(dst_h.at[pl.ds(wbase, SUP)], dw2[b], wsem)
        pltpu.async_copy(w_h.at[pl.ds(wbase, SUP)], ww2[b], wsem)

    def wait_window(b):
        pltpu.make_async_copy(gidx_h.at[pl.ds(0, SUP)], gw2[b], wsem).wait()
        pltpu.make_async_copy(dst_h.at[pl.ds(0, SUP)], dw2[b], wsem).wait()
        pltpu.make_async_copy(w_h.at[pl.ds(0, SUP)], ww2[b], wsem).wait()

    load_window(0, 0)
    plsc.subcore_barrier()

    def start_gather(gw, c, rows, gsem):
        pltpu.async_copy(y_h.at[gw.at[pl.ds(c * CH, CH)]], rows, gsem)

    def wait_gather(rows, gsem):
        pltpu.make_async_copy(y_h.at[gwA.at[pl.ds(0, CH)]], rows, gsem).wait()

    def wait_scatter(rows, dbuf, ssem):
        pltpu.make_async_copy(rows, acc.at[dbuf], ssem).wait()

    def process(ww, dw, c, rows, dbuf, ssem):
        for j in range(CH // 16):
            dbuf[pl.ds(j * 16, 16)] = dw[pl.ds(c * CH + j * 16, 16)]

        @plsc.parallel_loop(0, CH, 1, unroll=4)
        def _(e):
            wv = plsc.load_gather(ww, [_i32x16(0) + (c * CH + e)])
            rv = rows.at[e]
            for j in range(D // 16):
                sl = pl.ds(j * 16, 16)
                rv[sl] = rv[sl] * wv
        pltpu.async_copy(rows, acc.at[dbuf], ssem, add=True)

    for s in range(NWIN):
        b = s % 2
        gw = gw2[b]
        dw = dw2[b]
        ww = ww2[b]
        wait_window(b)
        if s + 1 < NWIN:
            load_window(s + 1, 1 - b)
        start_gather(gw, 0, rows0, gsem0)

        def pbody(k, _, gw=gw, dw=dw, ww=ww, first=(s == 0)):
            c0 = 2 * k
            wait_gather(rows0, gsem0)

            @pl.when(jnp.logical_or(k > 0, jnp.bool_(not first)))
            def _():
                wait_scatter(rows1, dbuf1, ssem1)
            start_gather(gw, c0 + 1, rows1, gsem1)
            process(ww, dw, c0, rows0, dbuf0, ssem0)

            wait_gather(rows1, gsem1)
            wait_scatter(rows0, dbuf0, ssem0)

            @pl.when(c0 + 2 < WCH)
            def _():
                start_gather(gw, c0 + 2, rows0, gsem0)
            process(ww, dw, c0 + 1, rows1, dbuf1, ssem1)
            return 0
        lax.fori_loop(0, WCH // 2, pbody, 0)

        wait_gather(rows0, gsem0)
        process(ww, dw, WCH - 1, rows0, dbuf0, ssem0)
        wait_scatter(rows0, dbuf0, ssem0)

    wait_scatter(rows1, dbuf1, ssem1)
    plsc.subcore_barrier()
    sl = pl.ds(sid * NPW, NPW)
    pltpu.sync_copy(acc.at[sl], out_h.at[cid].at[sl])


@functools.partial(
    pl.kernel,
    out_type=jax.ShapeDtypeStruct((2, P, D), jnp.float32),
    mesh=_mesh,
    scratch_types=[
        pltpu.VMEM((PPW,), jnp.int32),
        pltpu.VMEM((PPW, D), jnp.float32),
        pltpu.SemaphoreType.DMA,
    ],
    compiler_params=_sc_params,
)
def _sc_pairs(h_h, ia_h, ib_h, out_h, ibuf, rows, sem):
    cid = lax.axis_index("c")
    sid = lax.axis_index("s")
    wid = sid * NC + cid
    base = wid * PPW
    for side, idx_h in ((0, ia_h), (1, ib_h)):
        pltpu.sync_copy(idx_h.at[pl.ds(base, PPW)], ibuf)
        pltpu.async_copy(h_h.at[ibuf], rows, sem).wait()
        pltpu.sync_copy(rows, out_h.at[side].at[pl.ds(base, PPW)])


BN = 1000


def _mm9_first_body(x_ref, w_ref, root_ref, y_ref):
    r = pl.program_id(1)
    h = x_ref[...]
    out = jnp.dot(h, w_ref[0], preferred_element_type=jnp.float32)

    @pl.when(r == 0)
    def _():
        root_ref[...] = out

    @pl.when(r > 0)
    def _():
        y_ref[0] = out


def _mm9_next_body(root_ref, b_ref, msg_ref, w_ref, nroot_ref, y_ref):
    r = pl.program_id(1)
    h = jax.nn.relu(root_ref[...] + b_ref[...] + msg_ref[0] + msg_ref[1])
    out = jnp.dot(h, w_ref[0], preferred_element_type=jnp.float32)

    @pl.when(r == 0)
    def _():
        nroot_ref[...] = out

    @pl.when(r > 0)
    def _():
        y_ref[0] = out


def _mm9_out_specs():
    return [
        pl.BlockSpec((BN, D), lambda i, r: (i, 0)),
        pl.BlockSpec((1, BN, D), lambda i, r: (jnp.maximum(r - 1, 0), i, 0)),
    ]


def _mm9_out_shapes():
    return [
        jax.ShapeDtypeStruct((N, D), jnp.float32),
        jax.ShapeDtypeStruct((R, N, D), jnp.float32),
    ]


_mm9_first = pl.pallas_call(
    _mm9_first_body,
    grid=(N // BN, R + 1),
    in_specs=[
        pl.BlockSpec((BN, D), lambda i, r: (i, 0)),
        pl.BlockSpec((1, D, D), lambda i, r: (r, 0, 0)),
    ],
    out_specs=_mm9_out_specs(),
    out_shape=_mm9_out_shapes(),
)

_mm9_next = pl.pallas_call(
    _mm9_next_body,
    grid=(N // BN, R + 1),
    in_specs=[
        pl.BlockSpec((BN, D), lambda i, r: (i, 0)),
        pl.BlockSpec((1, D), lambda i, r: (0, 0)),
        pl.BlockSpec((2, BN, D), lambda i, r: (0, i, 0)),
        pl.BlockSpec((1, D, D), lambda i, r: (r, 0, 0)),
    ],
    out_specs=_mm9_out_specs(),
    out_shape=_mm9_out_shapes(),
)


def _combine_body(root_ref, b_ref, msg_ref, h_ref):
    h_ref[...] = jax.nn.relu(root_ref[...] + b_ref[...] + msg_ref[0] + msg_ref[1])


_combine = pl.pallas_call(
    _combine_body,
    grid=(N // BN,),
    in_specs=[
        pl.BlockSpec((BN, D), lambda i: (i, 0)),
        pl.BlockSpec((1, D), lambda i: (0, 0)),
        pl.BlockSpec((2, BN, D), lambda i: (0, i, 0)),
    ],
    out_specs=pl.BlockSpec((BN, D), lambda i: (i, 0)),
    out_shape=jax.ShapeDtypeStruct((N, D), jnp.float32),
)


BP = 256


def _bil_body(fa_ref, fb_ref, w2_ref, bb_ref,
              m1w_ref, m1b_ref, m2w_ref, m2b_ref, m3w_ref, m3b_ref,
              out_ref):
    fa = fa_ref[...]
    fb = fb_ref[...]
    t = jnp.dot(fa.astype(jnp.bfloat16), w2_ref[...],
                preferred_element_type=jnp.float32)
    t3 = t.reshape(BP, D, D)
    s = jnp.sum(t3 * fb[:, None, :], axis=2)
    x1 = jax.nn.relu(s + bb_ref[...])
    x2 = jax.nn.relu(jnp.dot(x1, m1w_ref[...],
                             preferred_element_type=jnp.float32) + m1b_ref[...])
    x3 = jax.nn.relu(jnp.dot(x2, m2w_ref[...],
                             preferred_element_type=jnp.float32) + m2b_ref[...])
    out_ref[...] = jnp.dot(x3, m3w_ref[...],
                           preferred_element_type=jnp.float32) + m3b_ref[...]


_bilmlp = pl.pallas_call(
    _bil_body,
    grid=(P // BP,),
    in_specs=[
        pl.BlockSpec((BP, D), lambda i: (i, 0)),
        pl.BlockSpec((BP, D), lambda i: (i, 0)),
        pl.BlockSpec((D, D * D), lambda i: (0, 0)),
        pl.BlockSpec((1, D), lambda i: (0, 0)),
        pl.BlockSpec((D, D), lambda i: (0, 0)),
        pl.BlockSpec((1, D), lambda i: (0, 0)),
        pl.BlockSpec((D, D), lambda i: (0, 0)),
        pl.BlockSpec((1, D), lambda i: (0, 0)),
        pl.BlockSpec((D, D), lambda i: (0, 0)),
        pl.BlockSpec((1, D), lambda i: (0, 0)),
    ],
    out_specs=pl.BlockSpec((BP, D), lambda i: (i, 0)),
    out_shape=jax.ShapeDtypeStruct((P, D), jnp.float32),
)


def kernel(x, edge_index, edge_type, node_type, edge_filter,
           c1_Wr, c1_Wroot, c1_b, c2_Wr, c2_Wroot, c2_b,
           c3_Wr, c3_Wroot, c3_b,
           bil_W, bil_b, m1_W, m1_b, m2_W, m2_b, m3_W, m3_b):
    src = edge_index[0]
    dst = edge_index[1]

    zcnt = jnp.zeros((NRR, 128), jnp.int32)
    gidx, w, ia, ib = _sc_prep(src, dst, edge_type, node_type,
                               edge_filter[0], edge_filter[1], zcnt)

    zrows = jnp.zeros((NPW, D), jnp.float32)

    w1 = jnp.concatenate([c1_Wroot[None], c1_Wr], axis=0)
    w2 = jnp.concatenate([c2_Wroot[None], c2_Wr], axis=0)
    w3 = jnp.concatenate([c3_Wroot[None], c3_Wr], axis=0)

    root1, y1 = _mm9_first(x, w1)
    m1 = _sc_scatter(y1.reshape(R * N, D), gidx, dst, w, zrows)
    root2, y2 = _mm9_next(root1, c1_b.reshape(1, D), m1, w2)
    m2 = _sc_scatter(y2.reshape(R * N, D), gidx, dst, w, zrows)
    root3, y3 = _mm9_next(root2, c2_b.reshape(1, D), m2, w3)
    m3 = _sc_scatter(y3.reshape(R * N, D), gidx, dst, w, zrows)
    h3 = _combine(root3, c3_b.reshape(1, D), m3)

    pairs = _sc_pairs(h3, ia, ib)

    w2t = jnp.transpose(bil_W, (1, 0, 2)).reshape(D, D * D).astype(jnp.bfloat16)
    return _bilmlp(pairs[0], pairs[1], w2t,
                   bil_b.reshape(1, D),
                   m1_W, m1_b.reshape(1, D),
                   m2_W, m2_b.reshape(1, D),
                   m3_W, m3_b.reshape(1, D))
